# baseline (speedup 1.0000x reference)
import functools

import jax
import jax.numpy as jnp
import numpy as np
from jax import lax
from jax.experimental import pallas as pl
from jax.experimental.pallas import tpu as pltpu
from jax.experimental.pallas import tpu_sc as plsc

N = 10000
E = 320000
D = 128
H = 8
DH = 16
KV_D = 2 * D

NCORES = 2
SUB = 16
NW = NCORES * SUB
EPW = E // NW
BE = 40
NBLK = EPW // BE
RPS = 640
RPS_LAST = N - RPS * (SUB - 1)
NZ = 640
ZRPS = NZ // SUB


def _proj_nodes_body(h_ref, wq_ref, bq_ref, wk_ref, bk_ref, wv_ref, bv_ref,
                     qh_ref, kv_ref):
    hb = h_ref[...]
    qh_ref[...] = (jnp.dot(hb, wq_ref[...], preferred_element_type=jnp.float32)
                   + bq_ref[...])
    k = jnp.dot(hb, wk_ref[...], preferred_element_type=jnp.float32) + bk_ref[...]
    v = jnp.dot(hb, wv_ref[...], preferred_element_type=jnp.float32) + bv_ref[...]
    kv_ref[...] = jnp.concatenate([k, v], axis=1)


def _proj_nodes(h, WQ, bQ, WK, bK, WV, bV):
    blk = 2000
    grid = N // blk
    w_spec = pl.BlockSpec((D, D), lambda i: (0, 0))
    b_spec = pl.BlockSpec((1, D), lambda i: (0, 0))
    return pl.pallas_call(
        _proj_nodes_body,
        grid=(grid,),
        in_specs=[pl.BlockSpec((blk, D), lambda i: (i, 0)),
                  w_spec, b_spec, w_spec, b_spec, w_spec, b_spec],
        out_specs=[pl.BlockSpec((blk, D), lambda i: (i, 0)),
                   pl.BlockSpec((blk, KV_D), lambda i: (i, 0))],
        out_shape=[jax.ShapeDtypeStruct((N, D), jnp.float32),
                   jax.ShapeDtypeStruct((N, KV_D), jnp.float32)],
    )(h, WQ, bQ.reshape(1, D), WK, bK.reshape(1, D), WV, bV.reshape(1, D))


def _proj_edges_body(e_ref, w_ref, b_ref, qe_ref):
    qe_ref[...] = (jnp.dot(e_ref[...], w_ref[...],
                           preferred_element_type=jnp.float32) + b_ref[...])


def _proj_edges(e, WQe, bQe):
    blk = 4000
    grid = E // blk
    return pl.pallas_call(
        _proj_edges_body,
        grid=(grid,),
        in_specs=[pl.BlockSpec((blk, D), lambda i: (i, 0)),
                  pl.BlockSpec((D, D), lambda i: (0, 0)),
                  pl.BlockSpec((1, D), lambda i: (0, 0))],
        out_specs=pl.BlockSpec((blk, D), lambda i: (i, 0)),
        out_shape=jax.ShapeDtypeStruct((E, D), jnp.float32),
    )(e, WQe, bQe.reshape(1, D))


def _sc_body(qh_hbm, kv_hbm, qe_hbm, src_hbm, dst_hbm, zw_hbm,
             eout_hbm, wv_parts, zp_parts,
             src_v, dst_v, dstp_v, zt_v, zh_v, kv_buf, q_buf, qe_buf,
             eout_buf, acc_buf, z_buf, wv_sh, zp_sh, sem1, sem2):
    c = lax.axis_index("c")
    s = lax.axis_index("s")
    wid = c * SUB + s
    base0 = wid * EPW
    lane = lax.iota(jnp.int32, 16)

    row0 = s * RPS
    zrow0 = s * ZRPS

    pltpu.sync_copy(zw_hbm.at[pl.ds(0, ZRPS)], zp_sh.at[pl.ds(zrow0, ZRPS)])

    @pl.when(s < SUB - 1)
    def _():
        pltpu.sync_copy(zw_hbm, wv_sh.at[pl.ds(row0, RPS)])

    @pl.when(s == SUB - 1)
    def _():
        pltpu.sync_copy(zw_hbm.at[pl.ds(0, RPS_LAST)],
                        wv_sh.at[pl.ds(row0, RPS_LAST)])

    plsc.subcore_barrier()

    def blk(b, carry):
        base = base0 + b * BE
        pltpu.sync_copy(src_hbm.at[pl.ds(base, BE)], src_v)
        pltpu.sync_copy(dst_hbm.at[pl.ds(base, BE)], dst_v)
        kv_cp = pltpu.async_copy(kv_hbm.at[src_v], kv_buf, sem1)
        q_cp = pltpu.async_copy(qh_hbm.at[dst_v], q_buf, sem2)
        pltpu.sync_copy(qe_hbm.at[pl.ds(base, BE)], qe_buf)
        for gch in sorted(set(min(g, BE - 16) for g in range(0, BE, 16))):
            dv = dst_v[pl.ds(gch, 16)]
            dstp_v[pl.ds(gch, 16)] = jnp.right_shift(dv, 4)
            zt_v[pl.ds(gch, 16)] = jnp.right_shift(dv & 15, 1)
            zh_v[pl.ds(gch, 16)] = dv & 1
        kv_cp.wait()
        q_cp.wait()

        @plsc.parallel_loop(0, BE, 1, unroll=1)
        def edge(i):
            z_lo = jnp.zeros((16,), jnp.float32)
            z_hi = jnp.zeros((16,), jnp.float32)
            for r in range(H):
                off = r * DH
                k = kv_buf[i, pl.ds(off, DH)]
                q = q_buf[i, pl.ds(off, DH)]
                g = qe_buf[i, pl.ds(off, DH)]
                sc = k * q * g * 0.25
                eout_buf[i, pl.ds(off, DH)] = sc
                t = plsc.cumsum(sc)[15]
                sv = jnp.exp(jnp.clip(jnp.broadcast_to(t, (16,)), -5.0, 5.0))
                v = kv_buf[i, pl.ds(D + off, DH)]
                acc_buf[i, pl.ds(off, DH)] = v * sv
                z_lo = jnp.where(lane == r, sv, z_lo)
                z_hi = jnp.where(lane == r + 8, sv, z_hi)
            tv = jnp.broadcast_to(zt_v[pl.ds(i, 16)][0], (16,))
            hv = jnp.broadcast_to(zh_v[pl.ds(i, 16)][0], (16,))
            val = jnp.where(hv == 1, z_hi, z_lo)
            for tt in range(8):
                z_buf[i, pl.ds(tt * DH, DH)] = jnp.where(
                    tv == tt, val, jnp.zeros((16,), jnp.float32))

        pltpu.sync_copy(eout_buf, eout_hbm.at[pl.ds(base, BE)])
        pltpu.sync_copy(acc_buf, wv_sh.at[dst_v], add=True)
        pltpu.sync_copy(z_buf, zp_sh.at[dstp_v], add=True)
        return carry

    lax.fori_loop(0, NBLK, blk, 0)

    plsc.subcore_barrier()

    pltpu.sync_copy(zp_sh.at[pl.ds(zrow0, ZRPS)],
                    zp_parts.at[c, pl.ds(zrow0, ZRPS)])

    @pl.when(s < SUB - 1)
    def _():
        pltpu.sync_copy(wv_sh.at[pl.ds(row0, RPS)],
                        wv_parts.at[c, pl.ds(row0, RPS)])

    @pl.when(s == SUB - 1)
    def _():
        pltpu.sync_copy(wv_sh.at[pl.ds(row0, RPS_LAST)],
                        wv_parts.at[c, pl.ds(row0, RPS_LAST)])


def _sc_attention(qh, kv, qe, src, dst, zw):
    mesh = plsc.VectorSubcoreMesh(core_axis_name="c", subcore_axis_name="s")
    f = pl.kernel(
        _sc_body,
        out_type=(jax.ShapeDtypeStruct((E, D), jnp.float32),
                  jax.ShapeDtypeStruct((NCORES, N, D), jnp.float32),
                  jax.ShapeDtypeStruct((NCORES, NZ, D), jnp.float32)),
        mesh=mesh,
        compiler_params=pltpu.CompilerParams(needs_layout_passes=False),
        scratch_types=[
            pltpu.VMEM((BE,), jnp.int32),
            pltpu.VMEM((BE,), jnp.int32),
            pltpu.VMEM((BE,), jnp.int32),
            pltpu.VMEM((BE + 16,), jnp.int32),
            pltpu.VMEM((BE + 16,), jnp.int32),
            pltpu.VMEM((BE, KV_D), jnp.float32),
            pltpu.VMEM((BE, D), jnp.float32),
            pltpu.VMEM((BE, D), jnp.float32),
            pltpu.VMEM((BE, D), jnp.float32),
            pltpu.VMEM((BE, D), jnp.float32),
            pltpu.VMEM((BE, D), jnp.float32),
            pltpu.VMEM_SHARED((N, D), jnp.float32),
            pltpu.VMEM_SHARED((NZ, D), jnp.float32),
            pltpu.SemaphoreType.DMA,
            pltpu.SemaphoreType.DMA,
        ],
    )
    return f(qh, kv, qe, src, dst, zw)


def _final_body(wv_ref, z_ref, r_ref, out_ref):
    wv = wv_ref[0] + wv_ref[1]
    z = z_ref[0] + z_ref[1]
    zrep = jnp.dot(z, r_ref[...], preferred_element_type=jnp.float32)
    out_ref[...] = wv / (zrep + 1e-6)


def _final_combine(wv_parts, z8_parts, rep):
    blk = 2000
    grid = N // blk
    return pl.pallas_call(
        _final_body,
        grid=(grid,),
        in_specs=[pl.BlockSpec((NCORES, blk, D), lambda i: (0, i, 0)),
                  pl.BlockSpec((NCORES, blk, H), lambda i: (0, i, 0)),
                  pl.BlockSpec((H, D), lambda i: (0, 0))],
        out_specs=pl.BlockSpec((blk, D), lambda i: (i, 0)),
        out_shape=jax.ShapeDtypeStruct((N, D), jnp.float32),
    )(wv_parts, z8_parts, rep)


_REP = np.zeros((H, D), dtype=np.float32)
for _r in range(H):
    _REP[_r, _r * DH:(_r + 1) * DH] = 1.0


@jax.jit
def kernel(h, e, edge_index, WQ, bQ, WK, bK, WV, bV, WQe, bQe):
    qh, kv = _proj_nodes(h, WQ, bQ, WK, bK, WV, bV)
    qe = _proj_edges(e, WQe, bQe)
    src = edge_index[0]
    dst = edge_index[1]
    zw = jnp.zeros((RPS, D), jnp.float32)
    eout, wv_parts, zp_parts = _sc_attention(qh, kv, qe, src, dst, zw)
    z8_parts = zp_parts.reshape(NCORES, NZ * DH, H)[:, :N, :]
    h_out = _final_combine(wv_parts, z8_parts, jnp.asarray(_REP))
    return h_out.reshape(N, H, DH), eout.reshape(E, H, DH)


# speedup vs baseline: 1.0563x; 1.0563x over previous
import functools

import jax
import jax.numpy as jnp
import numpy as np
from jax import lax
from jax.experimental import pallas as pl
from jax.experimental.pallas import tpu as pltpu
from jax.experimental.pallas import tpu_sc as plsc

N = 10000
E = 320000
D = 128
H = 8
DH = 16
KV_D = 2 * D

NCORES = 2
SUB = 16
NW = NCORES * SUB
EPW = E // NW
BE = 40
NBLK = EPW // BE
RPS = 640
RPS_LAST = N - RPS * (SUB - 1)
NZ = 640
ZRPS = NZ // SUB


def _proj_nodes_body(h_ref, wq_ref, bq_ref, wk_ref, bk_ref, wv_ref, bv_ref,
                     qh_ref, kv_ref):
    hb = h_ref[...]
    qh_ref[...] = (jnp.dot(hb, wq_ref[...], preferred_element_type=jnp.float32)
                   + bq_ref[...])
    k = jnp.dot(hb, wk_ref[...], preferred_element_type=jnp.float32) + bk_ref[...]
    v = jnp.dot(hb, wv_ref[...], preferred_element_type=jnp.float32) + bv_ref[...]
    kv_ref[...] = jnp.concatenate([k, v], axis=1)


def _proj_nodes(h, WQ, bQ, WK, bK, WV, bV):
    blk = 2000
    grid = N // blk
    w_spec = pl.BlockSpec((D, D), lambda i: (0, 0))
    b_spec = pl.BlockSpec((1, D), lambda i: (0, 0))
    return pl.pallas_call(
        _proj_nodes_body,
        grid=(grid,),
        in_specs=[pl.BlockSpec((blk, D), lambda i: (i, 0)),
                  w_spec, b_spec, w_spec, b_spec, w_spec, b_spec],
        out_specs=[pl.BlockSpec((blk, D), lambda i: (i, 0)),
                   pl.BlockSpec((blk, KV_D), lambda i: (i, 0))],
        out_shape=[jax.ShapeDtypeStruct((N, D), jnp.float32),
                   jax.ShapeDtypeStruct((N, KV_D), jnp.float32)],
    )(h, WQ, bQ.reshape(1, D), WK, bK.reshape(1, D), WV, bV.reshape(1, D))


def _proj_edges_body(e_ref, w_ref, b_ref, qe_ref):
    qe_ref[...] = (jnp.dot(e_ref[...], w_ref[...],
                           preferred_element_type=jnp.float32) + b_ref[...])


def _proj_edges(e, WQe, bQe):
    blk = 4000
    grid = E // blk
    return pl.pallas_call(
        _proj_edges_body,
        grid=(grid,),
        in_specs=[pl.BlockSpec((blk, D), lambda i: (i, 0)),
                  pl.BlockSpec((D, D), lambda i: (0, 0)),
                  pl.BlockSpec((1, D), lambda i: (0, 0))],
        out_specs=pl.BlockSpec((blk, D), lambda i: (i, 0)),
        out_shape=jax.ShapeDtypeStruct((E, D), jnp.float32),
    )(e, WQe, bQe.reshape(1, D))


def _sc_body(qh_hbm, kv_hbm, qe_hbm, src_hbm, dst_hbm, zw_hbm,
             eout_hbm, wv_parts, zp_parts,
             src_v, dst_v, dstp_v, zt_v, zh_v, kv_buf, q_buf, qe_buf,
             eout_buf, acc_buf, z_buf, wv_sh, zp_sh, sem1, sem2):
    c = lax.axis_index("c")
    s = lax.axis_index("s")
    wid = c * SUB + s
    base0 = wid * EPW
    lane = lax.iota(jnp.int32, 16)

    row0 = s * RPS
    zrow0 = s * ZRPS

    pltpu.sync_copy(zw_hbm.at[pl.ds(0, ZRPS)], zp_sh.at[pl.ds(zrow0, ZRPS)])

    @pl.when(s < SUB - 1)
    def _():
        pltpu.sync_copy(zw_hbm, wv_sh.at[pl.ds(row0, RPS)])

    @pl.when(s == SUB - 1)
    def _():
        pltpu.sync_copy(zw_hbm.at[pl.ds(0, RPS_LAST)],
                        wv_sh.at[pl.ds(row0, RPS_LAST)])

    plsc.subcore_barrier()

    def blk(b, carry):
        base = base0 + b * BE
        pltpu.sync_copy(src_hbm.at[pl.ds(base, BE)], src_v)
        pltpu.sync_copy(dst_hbm.at[pl.ds(base, BE)], dst_v)
        kv_cp = pltpu.async_copy(kv_hbm.at[src_v], kv_buf, sem1)
        q_cp = pltpu.async_copy(qh_hbm.at[dst_v], q_buf, sem2)
        pltpu.sync_copy(qe_hbm.at[pl.ds(base, BE)], qe_buf)
        for gch in sorted(set(min(g, BE - 16) for g in range(0, BE, 16))):
            dv = dst_v[pl.ds(gch, 16)]
            dstp_v[pl.ds(gch, 16)] = jnp.right_shift(dv, 4)
            zt_v[pl.ds(gch, 16)] = jnp.right_shift(dv & 15, 1)
            zh_v[pl.ds(gch, 16)] = dv & 1
        kv_cp.wait()
        q_cp.wait()

        perm8 = (lane + 8) & 15

        @plsc.parallel_loop(0, BE, 1, unroll=2)
        def edge(i):
            z_lo = jnp.zeros((16,), jnp.float32)
            for r in range(H):
                off = r * DH
                k = kv_buf[i, pl.ds(off, DH)]
                q = q_buf[i, pl.ds(off, DH)]
                g = qe_buf[i, pl.ds(off, DH)]
                sc = k * q * g * 0.25
                eout_buf[i, pl.ds(off, DH)] = sc
                t = plsc.cumsum(sc)[15]
                sv = jnp.exp(jnp.clip(jnp.broadcast_to(t, (16,)), -5.0, 5.0))
                v = kv_buf[i, pl.ds(D + off, DH)]
                acc_buf[i, pl.ds(off, DH)] = v * sv
                z_lo = jnp.where(lane == r, sv, z_lo)
            tv = jnp.broadcast_to(zt_v[pl.ds(i, 16)][0], (16,))
            hv = jnp.broadcast_to(zh_v[pl.ds(i, 16)][0], (16,))
            z_hi = z_lo.at[perm8].get(mode="promise_in_bounds")
            val = jnp.where(hv == 1, z_hi, z_lo)
            for tt in range(8):
                z_buf[i, pl.ds(tt * DH, DH)] = jnp.where(
                    tv == tt, val, jnp.zeros((16,), jnp.float32))

        pltpu.sync_copy(eout_buf, eout_hbm.at[pl.ds(base, BE)])
        pltpu.sync_copy(acc_buf, wv_sh.at[dst_v], add=True)
        pltpu.sync_copy(z_buf, zp_sh.at[dstp_v], add=True)
        return carry

    lax.fori_loop(0, NBLK, blk, 0)

    plsc.subcore_barrier()

    pltpu.sync_copy(zp_sh.at[pl.ds(zrow0, ZRPS)],
                    zp_parts.at[c, pl.ds(zrow0, ZRPS)])

    @pl.when(s < SUB - 1)
    def _():
        pltpu.sync_copy(wv_sh.at[pl.ds(row0, RPS)],
                        wv_parts.at[c, pl.ds(row0, RPS)])

    @pl.when(s == SUB - 1)
    def _():
        pltpu.sync_copy(wv_sh.at[pl.ds(row0, RPS_LAST)],
                        wv_parts.at[c, pl.ds(row0, RPS_LAST)])


def _sc_attention(qh, kv, qe, src, dst, zw):
    mesh = plsc.VectorSubcoreMesh(core_axis_name="c", subcore_axis_name="s")
    f = pl.kernel(
        _sc_body,
        out_type=(jax.ShapeDtypeStruct((E, D), jnp.float32),
                  jax.ShapeDtypeStruct((NCORES, N, D), jnp.float32),
                  jax.ShapeDtypeStruct((NCORES, NZ, D), jnp.float32)),
        mesh=mesh,
        compiler_params=pltpu.CompilerParams(needs_layout_passes=False),
        scratch_types=[
            pltpu.VMEM((BE,), jnp.int32),
            pltpu.VMEM((BE,), jnp.int32),
            pltpu.VMEM((BE,), jnp.int32),
            pltpu.VMEM((BE + 16,), jnp.int32),
            pltpu.VMEM((BE + 16,), jnp.int32),
            pltpu.VMEM((BE, KV_D), jnp.float32),
            pltpu.VMEM((BE, D), jnp.float32),
            pltpu.VMEM((BE, D), jnp.float32),
            pltpu.VMEM((BE, D), jnp.float32),
            pltpu.VMEM((BE, D), jnp.float32),
            pltpu.VMEM((BE, D), jnp.float32),
            pltpu.VMEM_SHARED((N, D), jnp.float32),
            pltpu.VMEM_SHARED((NZ, D), jnp.float32),
            pltpu.SemaphoreType.DMA,
            pltpu.SemaphoreType.DMA,
        ],
    )
    return f(qh, kv, qe, src, dst, zw)


def _final_body(wv_ref, z_ref, r_ref, out_ref):
    wv = wv_ref[0] + wv_ref[1]
    z = z_ref[0] + z_ref[1]
    zrep = jnp.dot(z, r_ref[...], preferred_element_type=jnp.float32)
    out_ref[...] = wv / (zrep + 1e-6)


def _final_combine(wv_parts, z8_parts, rep):
    blk = 2000
    grid = N // blk
    return pl.pallas_call(
        _final_body,
        grid=(grid,),
        in_specs=[pl.BlockSpec((NCORES, blk, D), lambda i: (0, i, 0)),
                  pl.BlockSpec((NCORES, blk, H), lambda i: (0, i, 0)),
                  pl.BlockSpec((H, D), lambda i: (0, 0))],
        out_specs=pl.BlockSpec((blk, D), lambda i: (i, 0)),
        out_shape=jax.ShapeDtypeStruct((N, D), jnp.float32),
    )(wv_parts, z8_parts, rep)


_REP = np.zeros((H, D), dtype=np.float32)
for _r in range(H):
    _REP[_r, _r * DH:(_r + 1) * DH] = 1.0


@jax.jit
def kernel(h, e, edge_index, WQ, bQ, WK, bK, WV, bV, WQe, bQe):
    qh, kv = _proj_nodes(h, WQ, bQ, WK, bK, WV, bV)
    qe = _proj_edges(e, WQe, bQe)
    src = edge_index[0]
    dst = edge_index[1]
    zw = jnp.zeros((RPS, D), jnp.float32)
    eout, wv_parts, zp_parts = _sc_attention(qh, kv, qe, src, dst, zw)
    z8_parts = zp_parts.reshape(NCORES, NZ * DH, H)[:, :N, :]
    h_out = _final_combine(wv_parts, z8_parts, jnp.asarray(_REP))
    return h_out.reshape(N, H, DH), eout.reshape(E, H, DH)
